# Initial kernel scaffold; baseline (speedup 1.0000x reference)
import jax
import jax.numpy as jnp
from jax.experimental import pallas as pl


def kernel(x, sim_matrix, gates, experts_mask, temperature):
    raise NotImplementedError("write your pallas kernel here")


# speedup vs baseline: 1.3503x; 1.3503x over previous
import math

import jax
import jax.numpy as jnp
from jax.experimental import pallas as pl

TOKENS = 8192
MODEL_DIM = 2048
MAX_E = 64
CLAMP_MAX = math.log(1.0 / 0.01)

BT = 1024


def _gate_kernel(x_ref, sim_ref, gates_ref, mask_ref, temp_ref,
                 out_ref, topk_ref):
    scale = jnp.exp(jnp.minimum(temp_ref[0, 0], CLAMP_MAX))
    x = x_ref[...]
    rn = jnp.sqrt(jnp.sum(x * x, axis=1, keepdims=True))
    xn = x / jnp.maximum(rn, 1e-12)
    w = sim_ref[...]
    cn = jnp.sqrt(jnp.sum(w * w, axis=0, keepdims=True))
    wn = w / jnp.maximum(cn, 1e-12)
    s = jnp.dot(xn, wn, preferred_element_type=jnp.float32) * scale
    sig = jax.nn.sigmoid(s) * mask_ref[...]
    g = jax.nn.sigmoid(gates_ref[...] * scale)
    out = (sig - g > 0).astype(jnp.float32)
    out_ref[...] = out
    topk_ref[...] = jnp.sum(out, axis=1, keepdims=True).astype(jnp.int32)


def kernel(x, sim_matrix, gates, experts_mask, temperature):
    gates2 = gates.reshape(1, MAX_E)
    mask2 = experts_mask.reshape(1, MAX_E)
    temp2 = temperature.reshape(1, 1)
    grid = (TOKENS // BT,)
    logits_out, topk = pl.pallas_call(
        _gate_kernel,
        grid=grid,
        in_specs=[
            pl.BlockSpec((BT, MODEL_DIM), lambda i: (i, 0)),
            pl.BlockSpec((MODEL_DIM, MAX_E), lambda i: (0, 0)),
            pl.BlockSpec((1, MAX_E), lambda i: (0, 0)),
            pl.BlockSpec((1, MAX_E), lambda i: (0, 0)),
            pl.BlockSpec((1, 1), lambda i: (0, 0)),
        ],
        out_specs=[
            pl.BlockSpec((BT, MAX_E), lambda i: (i, 0)),
            pl.BlockSpec((BT, 1), lambda i: (i, 0)),
        ],
        out_shape=[
            jax.ShapeDtypeStruct((TOKENS, MAX_E), jnp.float32),
            jax.ShapeDtypeStruct((TOKENS, 1), jnp.int32),
        ],
    )(x, sim_matrix, gates2, mask2, temp2)
    return (logits_out, topk.reshape(TOKENS))
